# baseline (speedup 1.0000x reference)
import functools

import jax
import jax.numpy as jnp
from jax import lax
from jax.experimental import pallas as pl
from jax.experimental.pallas import tpu as pltpu
from jax.experimental.pallas import tpu_sc as plsc

_N = 10000
_E = 160000
_F = 256
_H = 50
_FC1 = 512
_NCLS = 10

_NP = 10240
_HP = 64
_NC = 2
_NS = 16
_NW = _NC * _NS
_EP = 163840
_ET = _EP // _NW
_CH = 128
_NCHUNK = _ET // _CH
_RT = _NP // _NS
_CLSP = 16
_EST = 256
_NCH2 = _ET // _EST
_ZR = 64

_sc_mesh = plsc.VectorSubcoreMesh(core_axis_name="c", subcore_axis_name="s")
_sc_params = pltpu.CompilerParams(use_tc_tiling_on_sc=False)


def _deg_body(dst_hbm, out0, out1, didx, ones_v, zed, accd):
    c = lax.axis_index("c")
    s = lax.axis_index("s")
    wid = c * _NS + s

    zero16 = jnp.zeros((16,), jnp.float32)
    one16 = jnp.ones((16,), jnp.float32)

    def _zero(i, carry):
        zed[pl.ds(i * 16, 16)] = zero16
        return carry

    lax.fori_loop(0, _RT // 16, _zero, 0)

    def _ones(i, carry):
        ones_v[pl.ds(i * 16, 16)] = one16
        return carry

    lax.fori_loop(0, _EST // 16, _ones, 0)
    pltpu.sync_copy(zed, accd.at[pl.ds(s * _RT, _RT)])
    plsc.subcore_barrier()

    for j in range(_NCH2):
        pltpu.sync_copy(dst_hbm.at[pl.ds(wid * _ET + j * _EST, _EST)],
                        didx.at[j])

    def _chunk(j, carry):
        pltpu.sync_copy(ones_v, accd.at[didx.at[j]], add=True)
        return carry

    lax.fori_loop(0, _NCH2, _chunk, 0)
    plsc.subcore_barrier()

    @pl.when(c == 0)
    def _():
        pltpu.sync_copy(accd.at[pl.ds(s * _RT, _RT)], out0.at[pl.ds(s * _RT, _RT)])

    @pl.when(c == 1)
    def _():
        pltpu.sync_copy(accd.at[pl.ds(s * _RT, _RT)], out1.at[pl.ds(s * _RT, _RT)])


_deg_call = functools.partial(
    pl.kernel,
    out_type=(
        jax.ShapeDtypeStruct((_NP,), jnp.float32),
        jax.ShapeDtypeStruct((_NP,), jnp.float32),
    ),
    mesh=_sc_mesh,
    scratch_types=[
        pltpu.VMEM((_NCH2, _EST), jnp.int32),
        pltpu.VMEM((_EST,), jnp.float32),
        pltpu.VMEM((_RT,), jnp.float32),
        pltpu.VMEM_SHARED((_NP,), jnp.float32),
    ],
    compiler_params=_sc_params,
)(_deg_body)


def _edge_body(g_hbm, src_hbm, dst_hbm, out0, out1,
               sidx, didx, rows0, rows1, zbuf, g_sh, acc, sem0, sem1):
    c = lax.axis_index("c")
    s = lax.axis_index("s")
    wid = c * _NS + s

    pltpu.sync_copy(src_hbm.at[pl.ds(wid * _ET, _ET)], sidx)
    for j in range(_NCH2):
        pltpu.sync_copy(dst_hbm.at[pl.ds(wid * _ET + j * _EST, _EST)],
                        didx.at[j])
    pltpu.sync_copy(g_hbm.at[pl.ds(s * _RT, _RT)], g_sh.at[pl.ds(s * _RT, _RT)])

    zero16 = jnp.zeros((16,), jnp.float32)

    def _zero(i, carry):
        for j in range(_HP // 16):
            zbuf[i, pl.ds(j * 16, 16)] = zero16
        return carry

    lax.fori_loop(0, _ZR, _zero, 0)
    for k in range(_RT // _ZR):
        pltpu.sync_copy(zbuf, acc.at[pl.ds(s * _RT + k * _ZR, _ZR)])
    plsc.subcore_barrier()

    pltpu.async_copy(g_sh.at[sidx.at[pl.ds(0, _EST)]], rows0, sem0)

    def _pair(p, carry):
        e0 = 2 * p
        pltpu.async_copy(g_sh.at[sidx.at[pl.ds((e0 + 1) * _EST, _EST)]],
                         rows1, sem1)
        pltpu.make_async_copy(g_sh.at[sidx.at[pl.ds(e0 * _EST, _EST)]],
                              rows0, sem0).wait()
        pltpu.sync_copy(rows0, acc.at[didx.at[e0]], add=True)

        @pl.when(p < _NCH2 // 2 - 1)
        def _():
            pltpu.async_copy(g_sh.at[sidx.at[pl.ds((e0 + 2) * _EST, _EST)]],
                             rows0, sem0)

        pltpu.make_async_copy(g_sh.at[sidx.at[pl.ds((e0 + 1) * _EST, _EST)]],
                              rows1, sem1).wait()
        pltpu.sync_copy(rows1, acc.at[didx.at[e0 + 1]], add=True)
        return carry

    lax.fori_loop(0, _NCH2 // 2, _pair, 0)
    plsc.subcore_barrier()

    @pl.when(c == 0)
    def _():
        pltpu.sync_copy(acc.at[pl.ds(s * _RT, _RT)], out0.at[pl.ds(s * _RT, _RT)])

    @pl.when(c == 1)
    def _():
        pltpu.sync_copy(acc.at[pl.ds(s * _RT, _RT)], out1.at[pl.ds(s * _RT, _RT)])


_edge_call = functools.partial(
    pl.kernel,
    out_type=(
        jax.ShapeDtypeStruct((_NP, _HP), jnp.float32),
        jax.ShapeDtypeStruct((_NP, _HP), jnp.float32),
    ),
    mesh=_sc_mesh,
    scratch_types=[
        pltpu.VMEM((_ET,), jnp.int32),
        pltpu.VMEM((_NCH2, _EST), jnp.int32),
        pltpu.VMEM((_EST, _HP), jnp.float32),
        pltpu.VMEM((_EST, _HP), jnp.float32),
        pltpu.VMEM((_ZR, _HP), jnp.float32),
        pltpu.VMEM_SHARED((_NP, _HP), jnp.float32),
        pltpu.VMEM_SHARED((_NP, _HP), jnp.float32),
        pltpu.SemaphoreType.DMA,
        pltpu.SemaphoreType.DMA,
    ],
    compiler_params=_sc_params,
)(_edge_body)


_BR2 = 2048
_BRP = _BR2 // 2


def _elu(x):
    return jnp.where(x > 0, x, jnp.exp(jnp.minimum(x, 0.0)) - 1.0)


def _dinv_wide(d, n):
    dinv = lax.rsqrt(jnp.maximum(d + 1.0, 1e-12))
    return jnp.broadcast_to(dinv, (n, _HP))


def _mm1s_body(x_ref, w_ref, d_ref, g_ref):
    h = jnp.dot(x_ref[...], w_ref[...], preferred_element_type=jnp.float32)
    g_ref[...] = h * _dinv_wide(d_ref[...], _BR2)


def _mm1s(xp, w1p, degc):
    return pl.pallas_call(
        _mm1s_body,
        grid=(_NP // _BR2,),
        in_specs=[
            pl.BlockSpec((_BR2, _F), lambda i: (i, 0)),
            pl.BlockSpec((_F, _HP), lambda i: (0, 0)),
            pl.BlockSpec((_BR2, 1), lambda i: (i, 0)),
        ],
        out_specs=pl.BlockSpec((_BR2, _HP), lambda i: (i, 0)),
        out_shape=jax.ShapeDtypeStruct((_NP, _HP), jnp.float32),
    )(xp, w1p, degc)


def _layer_body(s0_ref, s1_ref, g_ref, d_ref, b_ref, w_ref, gout_ref):
    wide = _dinv_wide(d_ref[...], _BR2)
    su = s0_ref[...] + s1_ref[...] + g_ref[...]
    act = _elu(su * wide + b_ref[...])
    h2 = jnp.dot(act, w_ref[...], preferred_element_type=jnp.float32)
    gout_ref[...] = h2 * wide


def _layer(s0, s1, g, degc, bp, wp):
    return pl.pallas_call(
        _layer_body,
        grid=(_NP // _BR2,),
        in_specs=[
            pl.BlockSpec((_BR2, _HP), lambda i: (i, 0)),
            pl.BlockSpec((_BR2, _HP), lambda i: (i, 0)),
            pl.BlockSpec((_BR2, _HP), lambda i: (i, 0)),
            pl.BlockSpec((_BR2, 1), lambda i: (i, 0)),
            pl.BlockSpec((1, _HP), lambda i: (0, 0)),
            pl.BlockSpec((_HP, _HP), lambda i: (0, 0)),
        ],
        out_specs=pl.BlockSpec((_BR2, _HP), lambda i: (i, 0)),
        out_shape=jax.ShapeDtypeStruct((_NP, _HP), jnp.float32),
    )(s0, s1, g, degc, bp, wp)


def _head_body(s0_ref, s1_ref, g_ref, d_ref, b_ref,
               wf1_ref, bf1_ref, wf2_ref, bf2_ref, out_ref):
    wide = _dinv_wide(d_ref[...], _NP)
    su = s0_ref[...] + s1_ref[...] + g_ref[...]
    act = _elu(su * wide + b_ref[...])
    rows = lax.broadcasted_iota(jnp.int32, (_NP, 1), 0)
    act = jnp.where(rows < _N, act, 0.0)
    pooled = jnp.sum(act, axis=0, keepdims=True)
    z = jnp.dot(pooled, wf1_ref[...], preferred_element_type=jnp.float32)
    z = jnp.maximum(z + bf1_ref[...], 0.0)
    logits = jnp.dot(z, wf2_ref[...],
                     preferred_element_type=jnp.float32) + bf2_ref[...]
    m = jnp.max(logits, axis=-1, keepdims=True)
    e = jnp.exp(logits - m)
    out_ref[...] = e / jnp.sum(e, axis=-1, keepdims=True)


def _head(s0, s1, g, degc, bp, wf1p, bf1r, wf2p, bf2p):
    return pl.pallas_call(
        _head_body,
        in_specs=[
            pl.BlockSpec((_NP, _HP), lambda: (0, 0)),
            pl.BlockSpec((_NP, _HP), lambda: (0, 0)),
            pl.BlockSpec((_NP, _HP), lambda: (0, 0)),
            pl.BlockSpec((_NP, 1), lambda: (0, 0)),
            pl.BlockSpec((1, _HP), lambda: (0, 0)),
            pl.BlockSpec((_HP, _FC1), lambda: (0, 0)),
            pl.BlockSpec((1, _FC1), lambda: (0, 0)),
            pl.BlockSpec((_FC1, _CLSP), lambda: (0, 0)),
            pl.BlockSpec((1, _CLSP), lambda: (0, 0)),
        ],
        out_specs=pl.BlockSpec((1, _CLSP), lambda: (0, 0)),
        out_shape=jax.ShapeDtypeStruct((1, _CLSP), jnp.float32),
    )(s0, s1, g, degc, bp, wf1p, bf1r, wf2p, bf2p)


def kernel(x, edge_index, W1, b1, W2, b2, Wf1, bf1, Wf2, bf2):
    src = edge_index[0]
    dst = edge_index[1]
    pad_idx = jnp.full((_EP - _E,), _N, jnp.int32)
    srcp = jnp.concatenate([src, pad_idx])
    dstp = jnp.concatenate([dst, pad_idx])

    xp = jnp.pad(x, ((0, _NP - _N), (0, 0)))
    w1p = jnp.pad(W1, ((0, 0), (0, _HP - _H)))
    b1p = jnp.pad(b1, (0, _HP - _H)).reshape(1, _HP)
    w2p = jnp.pad(W2, ((0, _HP - _H), (0, _HP - _H)))
    b2p = jnp.pad(b2, (0, _HP - _H)).reshape(1, _HP)
    wf1p = jnp.pad(Wf1, ((0, _HP - _H), (0, 0)))
    bf1r = bf1.reshape(1, _FC1)
    wf2p = jnp.pad(Wf2, ((0, 0), (0, _CLSP - _NCLS)))
    bf2p = jnp.concatenate(
        [bf2, jnp.full((_CLSP - _NCLS,), -1e30, jnp.float32)]).reshape(1, _CLSP)

    deg0, deg1 = _deg_call(dstp)
    degc = (deg0 + deg1).reshape(_NP, 1)
    g1 = _mm1s(xp, w1p, degc)
    s10, s11 = _edge_call(g1, srcp, dstp)
    g2 = _layer(s10, s11, g1, degc, b1p, w2p)
    s20, s21 = _edge_call(g2, srcp, dstp)
    outp = _head(s20, s21, g2, degc, b2p, wf1p, bf1r, wf2p, bf2p)
    return outp[:, :_NCLS]


# speedup vs baseline: 1.1316x; 1.1316x over previous
import functools

import jax
import jax.numpy as jnp
from jax import lax
from jax.experimental import pallas as pl
from jax.experimental.pallas import tpu as pltpu
from jax.experimental.pallas import tpu_sc as plsc

_N = 10000
_E = 160000
_F = 256
_H = 50
_FC1 = 512
_NCLS = 10

_NP = 10240
_HP = 64
_NC = 2
_NS = 16
_NW = _NC * _NS
_EP = 163840
_ET = _EP // _NW
_CH = 128
_NCHUNK = _ET // _CH
_RT = _NP // _NS
_CLSP = 16
_ZR = 64

_sc_mesh = plsc.VectorSubcoreMesh(core_axis_name="c", subcore_axis_name="s")
_sc_params = pltpu.CompilerParams(use_tc_tiling_on_sc=False)


def _deg_body(dst_hbm, out0, out1, didx, ones_v, zed, accd):
    c = lax.axis_index("c")
    s = lax.axis_index("s")
    wid = c * _NS + s

    zero16 = jnp.zeros((16,), jnp.float32)
    one16 = jnp.ones((16,), jnp.float32)

    def _zero(i, carry):
        zed[pl.ds(i * 16, 16)] = zero16
        return carry

    lax.fori_loop(0, _RT // 16, _zero, 0)

    def _ones(i, carry):
        ones_v[pl.ds(i * 16, 16)] = one16
        return carry

    lax.fori_loop(0, _CH // 16, _ones, 0)
    pltpu.sync_copy(zed, accd.at[pl.ds(s * _RT, _RT)])
    plsc.subcore_barrier()

    pltpu.sync_copy(dst_hbm.at[pl.ds(wid * _NCHUNK, _NCHUNK)], didx)

    def _chunk(j, carry):
        pltpu.sync_copy(ones_v, accd.at[didx.at[j]], add=True)
        return carry

    lax.fori_loop(0, _NCHUNK, _chunk, 0)
    plsc.subcore_barrier()

    @pl.when(c == 0)
    def _():
        pltpu.sync_copy(accd.at[pl.ds(s * _RT, _RT)], out0.at[pl.ds(s * _RT, _RT)])

    @pl.when(c == 1)
    def _():
        pltpu.sync_copy(accd.at[pl.ds(s * _RT, _RT)], out1.at[pl.ds(s * _RT, _RT)])


_deg_call = functools.partial(
    pl.kernel,
    out_type=(
        jax.ShapeDtypeStruct((_NP,), jnp.float32),
        jax.ShapeDtypeStruct((_NP,), jnp.float32),
    ),
    mesh=_sc_mesh,
    scratch_types=[
        pltpu.VMEM((_NCHUNK, _CH), jnp.int32),
        pltpu.VMEM((_CH,), jnp.float32),
        pltpu.VMEM((_RT,), jnp.float32),
        pltpu.VMEM_SHARED((_NP,), jnp.float32),
    ],
    compiler_params=_sc_params,
)(_deg_body)


def _edge_body(g_hbm, src_hbm, dst_hbm, out0, out1,
               sidx, didx, rows0, rows1, rows2, rows3, zbuf, g_sh, acc,
               sem0, sem1, sem2, sem3):
    c = lax.axis_index("c")
    s = lax.axis_index("s")
    wid = c * _NS + s
    rows = (rows0, rows1, rows2, rows3)
    sems = (sem0, sem1, sem2, sem3)

    pltpu.sync_copy(src_hbm.at[pl.ds(wid * _NCHUNK, _NCHUNK)], sidx)
    pltpu.sync_copy(dst_hbm.at[pl.ds(wid * _NCHUNK, _NCHUNK)], didx)
    pltpu.sync_copy(g_hbm.at[pl.ds(s * _RT, _RT)], g_sh.at[pl.ds(s * _RT, _RT)])

    zero16 = jnp.zeros((16,), jnp.float32)

    def _zero(i, carry):
        for j in range(_HP // 16):
            zbuf[i, pl.ds(j * 16, 16)] = zero16
        return carry

    lax.fori_loop(0, _ZR, _zero, 0)
    for k in range(_RT // _ZR):
        pltpu.sync_copy(zbuf, acc.at[pl.ds(s * _RT + k * _ZR, _ZR)])
    plsc.subcore_barrier()

    for k in range(4):
        pltpu.async_copy(g_sh.at[sidx.at[k]], rows[k], sems[k])

    def _quad(p, carry):
        for k in range(4):
            j = 4 * p + k
            pltpu.make_async_copy(g_sh.at[sidx.at[j]], rows[k],
                                  sems[k]).wait()
            pltpu.sync_copy(rows[k], acc.at[didx.at[j]], add=True)

            @pl.when(j + 4 < _NCHUNK)
            def _():
                pltpu.async_copy(g_sh.at[sidx.at[j + 4]], rows[k], sems[k])
        return carry

    lax.fori_loop(0, _NCHUNK // 4, _quad, 0)
    plsc.subcore_barrier()

    @pl.when(c == 0)
    def _():
        pltpu.sync_copy(acc.at[pl.ds(s * _RT, _RT)], out0.at[pl.ds(s * _RT, _RT)])

    @pl.when(c == 1)
    def _():
        pltpu.sync_copy(acc.at[pl.ds(s * _RT, _RT)], out1.at[pl.ds(s * _RT, _RT)])


_edge_call = functools.partial(
    pl.kernel,
    out_type=(
        jax.ShapeDtypeStruct((_NP, _HP), jnp.float32),
        jax.ShapeDtypeStruct((_NP, _HP), jnp.float32),
    ),
    mesh=_sc_mesh,
    scratch_types=[
        pltpu.VMEM((_NCHUNK, _CH), jnp.int32),
        pltpu.VMEM((_NCHUNK, _CH), jnp.int32),
        pltpu.VMEM((_CH, _HP), jnp.float32),
        pltpu.VMEM((_CH, _HP), jnp.float32),
        pltpu.VMEM((_CH, _HP), jnp.float32),
        pltpu.VMEM((_CH, _HP), jnp.float32),
        pltpu.VMEM((_ZR, _HP), jnp.float32),
        pltpu.VMEM_SHARED((_NP, _HP), jnp.float32),
        pltpu.VMEM_SHARED((_NP, _HP), jnp.float32),
        pltpu.SemaphoreType.DMA,
        pltpu.SemaphoreType.DMA,
        pltpu.SemaphoreType.DMA,
        pltpu.SemaphoreType.DMA,
    ],
    compiler_params=_sc_params,
)(_edge_body)


_BR2 = 2048
_BRP = _BR2 // 2


def _elu(x):
    return jnp.where(x > 0, x, jnp.exp(jnp.minimum(x, 0.0)) - 1.0)


def _dinv_wide(d, n):
    dinv = lax.rsqrt(jnp.maximum(d + 1.0, 1e-12))
    return jnp.broadcast_to(dinv, (n, _HP))


def _mm1s_body(x_ref, w_ref, d_ref, g_ref):
    h = jnp.dot(x_ref[...], w_ref[...], preferred_element_type=jnp.float32)
    g_ref[...] = h * _dinv_wide(d_ref[...], _BR2)


def _mm1s(xp, w1p, degc):
    return pl.pallas_call(
        _mm1s_body,
        grid=(_NP // _BR2,),
        in_specs=[
            pl.BlockSpec((_BR2, _F), lambda i: (i, 0)),
            pl.BlockSpec((_F, _HP), lambda i: (0, 0)),
            pl.BlockSpec((_BR2, 1), lambda i: (i, 0)),
        ],
        out_specs=pl.BlockSpec((_BR2, _HP), lambda i: (i, 0)),
        out_shape=jax.ShapeDtypeStruct((_NP, _HP), jnp.float32),
    )(xp, w1p, degc)


def _layer_body(s0_ref, s1_ref, g_ref, d_ref, b_ref, w_ref, gout_ref):
    wide = _dinv_wide(d_ref[...], _BR2)
    su = s0_ref[...] + s1_ref[...] + g_ref[...]
    act = _elu(su * wide + b_ref[...])
    h2 = jnp.dot(act, w_ref[...], preferred_element_type=jnp.float32)
    gout_ref[...] = h2 * wide


def _layer(s0, s1, g, degc, bp, wp):
    return pl.pallas_call(
        _layer_body,
        grid=(_NP // _BR2,),
        in_specs=[
            pl.BlockSpec((_BR2, _HP), lambda i: (i, 0)),
            pl.BlockSpec((_BR2, _HP), lambda i: (i, 0)),
            pl.BlockSpec((_BR2, _HP), lambda i: (i, 0)),
            pl.BlockSpec((_BR2, 1), lambda i: (i, 0)),
            pl.BlockSpec((1, _HP), lambda i: (0, 0)),
            pl.BlockSpec((_HP, _HP), lambda i: (0, 0)),
        ],
        out_specs=pl.BlockSpec((_BR2, _HP), lambda i: (i, 0)),
        out_shape=jax.ShapeDtypeStruct((_NP, _HP), jnp.float32),
    )(s0, s1, g, degc, bp, wp)


def _head_body(s0_ref, s1_ref, g_ref, d_ref, b_ref,
               wf1_ref, bf1_ref, wf2_ref, bf2_ref, out_ref):
    wide = _dinv_wide(d_ref[...], _NP)
    su = s0_ref[...] + s1_ref[...] + g_ref[...]
    act = _elu(su * wide + b_ref[...])
    rows = lax.broadcasted_iota(jnp.int32, (_NP, 1), 0)
    act = jnp.where(rows < _N, act, 0.0)
    pooled = jnp.sum(act, axis=0, keepdims=True)
    z = jnp.dot(pooled, wf1_ref[...], preferred_element_type=jnp.float32)
    z = jnp.maximum(z + bf1_ref[...], 0.0)
    logits = jnp.dot(z, wf2_ref[...],
                     preferred_element_type=jnp.float32) + bf2_ref[...]
    m = jnp.max(logits, axis=-1, keepdims=True)
    e = jnp.exp(logits - m)
    out_ref[...] = e / jnp.sum(e, axis=-1, keepdims=True)


def _head(s0, s1, g, degc, bp, wf1p, bf1r, wf2p, bf2p):
    return pl.pallas_call(
        _head_body,
        in_specs=[
            pl.BlockSpec((_NP, _HP), lambda: (0, 0)),
            pl.BlockSpec((_NP, _HP), lambda: (0, 0)),
            pl.BlockSpec((_NP, _HP), lambda: (0, 0)),
            pl.BlockSpec((_NP, 1), lambda: (0, 0)),
            pl.BlockSpec((1, _HP), lambda: (0, 0)),
            pl.BlockSpec((_HP, _FC1), lambda: (0, 0)),
            pl.BlockSpec((1, _FC1), lambda: (0, 0)),
            pl.BlockSpec((_FC1, _CLSP), lambda: (0, 0)),
            pl.BlockSpec((1, _CLSP), lambda: (0, 0)),
        ],
        out_specs=pl.BlockSpec((1, _CLSP), lambda: (0, 0)),
        out_shape=jax.ShapeDtypeStruct((1, _CLSP), jnp.float32),
    )(s0, s1, g, degc, bp, wf1p, bf1r, wf2p, bf2p)


def kernel(x, edge_index, W1, b1, W2, b2, Wf1, bf1, Wf2, bf2):
    src = edge_index[0]
    dst = edge_index[1]
    pad_idx = jnp.full((_EP - _E,), _N, jnp.int32)
    srcp = jnp.concatenate([src, pad_idx]).reshape(_EP // _CH, _CH)
    dstp = jnp.concatenate([dst, pad_idx]).reshape(_EP // _CH, _CH)

    xp = jnp.pad(x, ((0, _NP - _N), (0, 0)))
    w1p = jnp.pad(W1, ((0, 0), (0, _HP - _H)))
    b1p = jnp.pad(b1, (0, _HP - _H)).reshape(1, _HP)
    w2p = jnp.pad(W2, ((0, _HP - _H), (0, _HP - _H)))
    b2p = jnp.pad(b2, (0, _HP - _H)).reshape(1, _HP)
    wf1p = jnp.pad(Wf1, ((0, _HP - _H), (0, 0)))
    bf1r = bf1.reshape(1, _FC1)
    wf2p = jnp.pad(Wf2, ((0, 0), (0, _CLSP - _NCLS)))
    bf2p = jnp.concatenate(
        [bf2, jnp.full((_CLSP - _NCLS,), -1e30, jnp.float32)]).reshape(1, _CLSP)

    deg0, deg1 = _deg_call(dstp)
    degc = (deg0 + deg1).reshape(_NP, 1)
    g1 = _mm1s(xp, w1p, degc)
    s10, s11 = _edge_call(g1, srcp, dstp)
    g2 = _layer(s10, s11, g1, degc, b1p, w2p)
    s20, s21 = _edge_call(g2, srcp, dstp)
    outp = _head(s20, s21, g2, degc, b2p, wf1p, bf1r, wf2p, bf2p)
    return outp[:, :_NCLS]


# speedup vs baseline: 1.2015x; 1.0618x over previous
import functools

import jax
import jax.numpy as jnp
from jax import lax
from jax.experimental import pallas as pl
from jax.experimental.pallas import tpu as pltpu
from jax.experimental.pallas import tpu_sc as plsc

_N = 10000
_E = 160000
_F = 256
_H = 50
_FC1 = 512
_NCLS = 10

_NP = 10240
_HP = 64
_NC = 2
_NS = 16
_NW = _NC * _NS
_EP = 163840
_ET = _EP // _NW
_CH = 128
_NCHUNK = _ET // _CH
_RT = _NP // _NS
_CLSP = 16
_ZR = 64

_sc_mesh = plsc.VectorSubcoreMesh(core_axis_name="c", subcore_axis_name="s")
_sc_params = pltpu.CompilerParams(use_tc_tiling_on_sc=False)


def _deg_body(dst_hbm, out0, out1, didx, ones_v, zed, accd):
    c = lax.axis_index("c")
    s = lax.axis_index("s")
    wid = c * _NS + s

    zero16 = jnp.zeros((16,), jnp.float32)
    one16 = jnp.ones((16,), jnp.float32)

    def _zero(i, carry):
        zed[pl.ds(i * 16, 16)] = zero16
        return carry

    lax.fori_loop(0, _RT // 16, _zero, 0)

    def _ones(i, carry):
        ones_v[pl.ds(i * 16, 16)] = one16
        return carry

    lax.fori_loop(0, _CH // 16, _ones, 0)
    pltpu.sync_copy(zed, accd.at[pl.ds(s * _RT, _RT)])
    plsc.subcore_barrier()

    pltpu.sync_copy(dst_hbm.at[pl.ds(wid * _NCHUNK, _NCHUNK)], didx)

    def _chunk(j, carry):
        pltpu.sync_copy(ones_v, accd.at[didx.at[j]], add=True)
        return carry

    lax.fori_loop(0, _NCHUNK, _chunk, 0)
    plsc.subcore_barrier()

    @pl.when(c == 0)
    def _():
        pltpu.sync_copy(accd.at[pl.ds(s * _RT, _RT)], out0.at[pl.ds(s * _RT, _RT)])

    @pl.when(c == 1)
    def _():
        pltpu.sync_copy(accd.at[pl.ds(s * _RT, _RT)], out1.at[pl.ds(s * _RT, _RT)])


_deg_call = functools.partial(
    pl.kernel,
    out_type=(
        jax.ShapeDtypeStruct((_NP,), jnp.float32),
        jax.ShapeDtypeStruct((_NP,), jnp.float32),
    ),
    mesh=_sc_mesh,
    scratch_types=[
        pltpu.VMEM((_NCHUNK, _CH), jnp.int32),
        pltpu.VMEM((_CH,), jnp.float32),
        pltpu.VMEM((_RT,), jnp.float32),
        pltpu.VMEM_SHARED((_NP,), jnp.float32),
    ],
    compiler_params=_sc_params,
)(_deg_body)


def _edge_body(g_hbm, src_hbm, dst_hbm, out0, out1,
               sidx, didx, rows0, rows1, rows2, rows3, zbuf, g_sh, acc,
               sem0, sem1, sem2, sem3, ssem0, ssem1, ssem2, ssem3):
    c = lax.axis_index("c")
    s = lax.axis_index("s")
    wid = c * _NS + s
    rows = (rows0, rows1, rows2, rows3)
    sems = (sem0, sem1, sem2, sem3)
    ssems = (ssem0, ssem1, ssem2, ssem3)

    pltpu.sync_copy(src_hbm.at[pl.ds(wid * _NCHUNK, _NCHUNK)], sidx)
    pltpu.sync_copy(dst_hbm.at[pl.ds(wid * _NCHUNK, _NCHUNK)], didx)
    pltpu.sync_copy(g_hbm.at[pl.ds(s * _RT, _RT)], g_sh.at[pl.ds(s * _RT, _RT)])

    zero16 = jnp.zeros((16,), jnp.float32)

    def _zero(i, carry):
        for j in range(_HP // 16):
            zbuf[i, pl.ds(j * 16, 16)] = zero16
        return carry

    lax.fori_loop(0, _ZR, _zero, 0)
    for k in range(_RT // _ZR):
        pltpu.sync_copy(zbuf, acc.at[pl.ds(s * _RT + k * _ZR, _ZR)])
    plsc.subcore_barrier()

    for k in range(3):
        pltpu.async_copy(g_sh.at[sidx.at[k]], rows[k], sems[k])

    def _quad(p, carry):
        for k in range(4):
            m = 4 * p + k
            kn = (k + 3) % 4
            pltpu.make_async_copy(g_sh.at[sidx.at[m]], rows[k],
                                  sems[k]).wait()
            pltpu.async_copy(rows[k], acc.at[didx.at[m]], ssems[k], add=True)

            @pl.when(m + 3 < _NCHUNK)
            def _():
                @pl.when(m >= 1)
                def _():
                    pltpu.make_async_copy(rows[kn], acc.at[didx.at[m - 1]],
                                          ssems[kn]).wait()

                pltpu.async_copy(g_sh.at[sidx.at[m + 3]], rows[kn], sems[kn])
        return carry

    lax.fori_loop(0, _NCHUNK // 4, _quad, 0)
    for k in range(4):
        pltpu.make_async_copy(rows[k], acc.at[didx.at[_NCHUNK - 4 + k]],
                              ssems[k]).wait()
    plsc.subcore_barrier()

    @pl.when(c == 0)
    def _():
        pltpu.sync_copy(acc.at[pl.ds(s * _RT, _RT)], out0.at[pl.ds(s * _RT, _RT)])

    @pl.when(c == 1)
    def _():
        pltpu.sync_copy(acc.at[pl.ds(s * _RT, _RT)], out1.at[pl.ds(s * _RT, _RT)])


_edge_call = functools.partial(
    pl.kernel,
    out_type=(
        jax.ShapeDtypeStruct((_NP, _HP), jnp.float32),
        jax.ShapeDtypeStruct((_NP, _HP), jnp.float32),
    ),
    mesh=_sc_mesh,
    scratch_types=[
        pltpu.VMEM((_NCHUNK, _CH), jnp.int32),
        pltpu.VMEM((_NCHUNK, _CH), jnp.int32),
        pltpu.VMEM((_CH, _HP), jnp.float32),
        pltpu.VMEM((_CH, _HP), jnp.float32),
        pltpu.VMEM((_CH, _HP), jnp.float32),
        pltpu.VMEM((_CH, _HP), jnp.float32),
        pltpu.VMEM((_ZR, _HP), jnp.float32),
        pltpu.VMEM_SHARED((_NP, _HP), jnp.float32),
        pltpu.VMEM_SHARED((_NP, _HP), jnp.float32),
        pltpu.SemaphoreType.DMA,
        pltpu.SemaphoreType.DMA,
        pltpu.SemaphoreType.DMA,
        pltpu.SemaphoreType.DMA,
        pltpu.SemaphoreType.DMA,
        pltpu.SemaphoreType.DMA,
        pltpu.SemaphoreType.DMA,
        pltpu.SemaphoreType.DMA,
    ],
    compiler_params=_sc_params,
)(_edge_body)


_BR2 = 2048
_BRP = _BR2 // 2


def _elu(x):
    return jnp.where(x > 0, x, jnp.exp(jnp.minimum(x, 0.0)) - 1.0)


def _dinv_wide(d, n):
    dinv = lax.rsqrt(jnp.maximum(d + 1.0, 1e-12))
    return jnp.broadcast_to(dinv, (n, _HP))


def _mm1s_body(x_ref, w_ref, d_ref, g_ref):
    h = jnp.dot(x_ref[...], w_ref[...], preferred_element_type=jnp.float32)
    g_ref[...] = h * _dinv_wide(d_ref[...], _BR2)


def _mm1s(xp, w1p, degc):
    return pl.pallas_call(
        _mm1s_body,
        grid=(_NP // _BR2,),
        in_specs=[
            pl.BlockSpec((_BR2, _F), lambda i: (i, 0)),
            pl.BlockSpec((_F, _HP), lambda i: (0, 0)),
            pl.BlockSpec((_BR2, 1), lambda i: (i, 0)),
        ],
        out_specs=pl.BlockSpec((_BR2, _HP), lambda i: (i, 0)),
        out_shape=jax.ShapeDtypeStruct((_NP, _HP), jnp.float32),
    )(xp, w1p, degc)


def _layer_body(s0_ref, s1_ref, g_ref, d_ref, b_ref, w_ref, gout_ref):
    wide = _dinv_wide(d_ref[...], _BR2)
    su = s0_ref[...] + s1_ref[...] + g_ref[...]
    act = _elu(su * wide + b_ref[...])
    h2 = jnp.dot(act, w_ref[...], preferred_element_type=jnp.float32)
    gout_ref[...] = h2 * wide


def _layer(s0, s1, g, degc, bp, wp):
    return pl.pallas_call(
        _layer_body,
        grid=(_NP // _BR2,),
        in_specs=[
            pl.BlockSpec((_BR2, _HP), lambda i: (i, 0)),
            pl.BlockSpec((_BR2, _HP), lambda i: (i, 0)),
            pl.BlockSpec((_BR2, _HP), lambda i: (i, 0)),
            pl.BlockSpec((_BR2, 1), lambda i: (i, 0)),
            pl.BlockSpec((1, _HP), lambda i: (0, 0)),
            pl.BlockSpec((_HP, _HP), lambda i: (0, 0)),
        ],
        out_specs=pl.BlockSpec((_BR2, _HP), lambda i: (i, 0)),
        out_shape=jax.ShapeDtypeStruct((_NP, _HP), jnp.float32),
    )(s0, s1, g, degc, bp, wp)


def _head_body(s0_ref, s1_ref, g_ref, d_ref, b_ref,
               wf1_ref, bf1_ref, wf2_ref, bf2_ref, out_ref):
    wide = _dinv_wide(d_ref[...], _NP)
    su = s0_ref[...] + s1_ref[...] + g_ref[...]
    act = _elu(su * wide + b_ref[...])
    rows = lax.broadcasted_iota(jnp.int32, (_NP, 1), 0)
    act = jnp.where(rows < _N, act, 0.0)
    pooled = jnp.sum(act, axis=0, keepdims=True)
    z = jnp.dot(pooled, wf1_ref[...], preferred_element_type=jnp.float32)
    z = jnp.maximum(z + bf1_ref[...], 0.0)
    logits = jnp.dot(z, wf2_ref[...],
                     preferred_element_type=jnp.float32) + bf2_ref[...]
    m = jnp.max(logits, axis=-1, keepdims=True)
    e = jnp.exp(logits - m)
    out_ref[...] = e / jnp.sum(e, axis=-1, keepdims=True)


def _head(s0, s1, g, degc, bp, wf1p, bf1r, wf2p, bf2p):
    return pl.pallas_call(
        _head_body,
        in_specs=[
            pl.BlockSpec((_NP, _HP), lambda: (0, 0)),
            pl.BlockSpec((_NP, _HP), lambda: (0, 0)),
            pl.BlockSpec((_NP, _HP), lambda: (0, 0)),
            pl.BlockSpec((_NP, 1), lambda: (0, 0)),
            pl.BlockSpec((1, _HP), lambda: (0, 0)),
            pl.BlockSpec((_HP, _FC1), lambda: (0, 0)),
            pl.BlockSpec((1, _FC1), lambda: (0, 0)),
            pl.BlockSpec((_FC1, _CLSP), lambda: (0, 0)),
            pl.BlockSpec((1, _CLSP), lambda: (0, 0)),
        ],
        out_specs=pl.BlockSpec((1, _CLSP), lambda: (0, 0)),
        out_shape=jax.ShapeDtypeStruct((1, _CLSP), jnp.float32),
    )(s0, s1, g, degc, bp, wf1p, bf1r, wf2p, bf2p)


def kernel(x, edge_index, W1, b1, W2, b2, Wf1, bf1, Wf2, bf2):
    src = edge_index[0]
    dst = edge_index[1]
    pad_idx = jnp.full((_EP - _E,), _N, jnp.int32)
    srcp = jnp.concatenate([src, pad_idx]).reshape(_EP // _CH, _CH)
    dstp = jnp.concatenate([dst, pad_idx]).reshape(_EP // _CH, _CH)

    xp = jnp.pad(x, ((0, _NP - _N), (0, 0)))
    w1p = jnp.pad(W1, ((0, 0), (0, _HP - _H)))
    b1p = jnp.pad(b1, (0, _HP - _H)).reshape(1, _HP)
    w2p = jnp.pad(W2, ((0, _HP - _H), (0, _HP - _H)))
    b2p = jnp.pad(b2, (0, _HP - _H)).reshape(1, _HP)
    wf1p = jnp.pad(Wf1, ((0, _HP - _H), (0, 0)))
    bf1r = bf1.reshape(1, _FC1)
    wf2p = jnp.pad(Wf2, ((0, 0), (0, _CLSP - _NCLS)))
    bf2p = jnp.concatenate(
        [bf2, jnp.full((_CLSP - _NCLS,), -1e30, jnp.float32)]).reshape(1, _CLSP)

    deg0, deg1 = _deg_call(dstp)
    degc = (deg0 + deg1).reshape(_NP, 1)
    g1 = _mm1s(xp, w1p, degc)
    s10, s11 = _edge_call(g1, srcp, dstp)
    g2 = _layer(s10, s11, g1, degc, b1p, w2p)
    s20, s21 = _edge_call(g2, srcp, dstp)
    outp = _head(s20, s21, g2, degc, b2p, wf1p, bf1r, wf2p, bf2p)
    return outp[:, :_NCLS]
